# Initial kernel scaffold; baseline (speedup 1.0000x reference)
import jax
import jax.numpy as jnp
from jax.experimental import pallas as pl


def kernel(x, start, emb_weight):
    raise NotImplementedError("write your pallas kernel here")


# speedup vs baseline: 2.8503x; 2.8503x over previous
import jax
import jax.numpy as jnp
from jax.experimental import pallas as pl
from jax.experimental.pallas import tpu as pltpu


def _body(start_ref, emb_ref, x_ref, o_ref, *, bs, max_len):
    i = pl.program_id(0)
    base = pl.multiple_of(jnp.clip(start_ref[0] + i * bs, 0, max_len - bs), 8)
    o_ref[...] = x_ref[...] + emb_ref[pl.ds(base, bs), :][None]


def kernel(x, start, emb_weight):
    B, S, D = x.shape
    max_len = emb_weight.shape[0]
    bs = 512
    start_arr = jnp.asarray(start, jnp.int32).reshape(1)

    import functools
    body = functools.partial(_body, bs=bs, max_len=max_len)

    out = pl.pallas_call(
        body,
        grid=(S // bs, B),
        in_specs=[
            pl.BlockSpec(memory_space=pltpu.SMEM),
            pl.BlockSpec((max_len, D), lambda i, b: (0, 0)),
            pl.BlockSpec((1, bs, D), lambda i, b: (b, i, 0)),
        ],
        out_specs=pl.BlockSpec((1, bs, D), lambda i, b: (b, i, 0)),
        out_shape=jax.ShapeDtypeStruct((B, S, D), x.dtype),
    )(start_arr, emb_weight, x)
    return out
